# baseline (speedup 1.0000x reference)
import functools

import jax
import jax.numpy as jnp
from jax import lax
from jax.experimental import pallas as pl
from jax.experimental.pallas import tpu as pltpu
from jax.experimental.pallas import tpu_sc as plsc

_N = 10000
_NE = 320000
_DIN = 128
_DH = 16
_DOUT = 128

_NPAD = 10240
_STRIPE = _NPAD // 16
_NCORES = 2
_NSUB = 16
_NW = _NCORES * _NSUB
_CHUNK = 128
_CB = 8
_KS = 10
_KSB = _KS + 2
_K = _KS * _CB

_BM = 2048

_vmesh = plsc.VectorSubcoreMesh(core_axis_name="c", subcore_axis_name="s")


@functools.partial(
    pl.kernel,
    mesh=_vmesh,
    out_type=jax.ShapeDtypeStruct((_NCORES, _NPAD, _DH), jnp.float32),
    scratch_types=[
        pltpu.VMEM((_KSB, _CB * _CHUNK), jnp.int32),
        pltpu.VMEM((_KS, _CB * _CHUNK), jnp.int32),
        pltpu.VMEM((_CB * _CHUNK, _DH), jnp.float32),
        pltpu.VMEM((_CB * _CHUNK, _DH), jnp.float32),
        pltpu.VMEM((_STRIPE, _DH), jnp.float32),
        pltpu.VMEM_SHARED((_NPAD, _DH), jnp.float32),
        pltpu.SemaphoreType.DMA,
        pltpu.SemaphoreType.DMA,
    ],
    compiler_params=pltpu.CompilerParams(use_tc_tiling_on_sc=False),
)
def _prop(y_hbm, src_hbm, dst_hbm, out_hbm,
          src_v, dst_v, rows0_v, rows1_v, zero_v, acc_sh, sem0, sem1):
    c = lax.axis_index("c")
    s = lax.axis_index("s")
    w = c * _NSUB + s

    @pl.loop(0, _STRIPE)
    def _(i):
        zero_v[i, :] = jnp.zeros((_DH,), jnp.float32)

    pltpu.async_copy(src_hbm.at[w], src_v, sem0).wait()
    pltpu.async_copy(dst_hbm.at[w], dst_v, sem0).wait()
    pltpu.sync_copy(zero_v, acc_sh.at[pl.ds(s * _STRIPE, _STRIPE)])
    plsc.subcore_barrier()

    pltpu.async_copy(y_hbm.at[src_v.at[0]], rows0_v, sem0)
    pltpu.async_copy(y_hbm.at[src_v.at[1]], rows1_v, sem1)

    @pl.loop(0, _KS, step=2)
    def _(j):
        pltpu.make_async_copy(y_hbm.at[src_v.at[j]], rows0_v, sem0).wait()
        pltpu.sync_copy(rows0_v, acc_sh.at[dst_v.at[j]], add=True)
        pltpu.async_copy(y_hbm.at[src_v.at[j + 2]], rows0_v, sem0)
        pltpu.make_async_copy(y_hbm.at[src_v.at[j + 1]], rows1_v, sem1).wait()
        pltpu.sync_copy(rows1_v, acc_sh.at[dst_v.at[j + 1]], add=True)
        pltpu.async_copy(y_hbm.at[src_v.at[j + 3]], rows1_v, sem1)

    pltpu.make_async_copy(y_hbm.at[src_v.at[_KS]], rows0_v, sem0).wait()
    pltpu.make_async_copy(y_hbm.at[src_v.at[_KS + 1]], rows1_v, sem1).wait()

    plsc.subcore_barrier()
    pltpu.sync_copy(acc_sh.at[pl.ds(s * _STRIPE, _STRIPE)],
                    out_hbm.at[c, pl.ds(s * _STRIPE, _STRIPE)])


@functools.partial(
    pl.kernel,
    mesh=_vmesh,
    out_type=jax.ShapeDtypeStruct((_NCORES, _NPAD, _DH), jnp.float32),
    scratch_types=[
        pltpu.VMEM((_KS, _CB * _CHUNK), jnp.int32),
        pltpu.VMEM((_CB * _CHUNK, _DH), jnp.float32),
        pltpu.VMEM((_STRIPE, _DH), jnp.float32),
        pltpu.VMEM_SHARED((_NPAD, _DH), jnp.float32),
        pltpu.SemaphoreType.DMA,
    ],
    compiler_params=pltpu.CompilerParams(use_tc_tiling_on_sc=False),
)
def _deg(dst_hbm, out_hbm, dst_v, ones_v, zero_v, acc_sh, sem):
    c = lax.axis_index("c")
    s = lax.axis_index("s")
    w = c * _NSUB + s

    @pl.loop(0, _STRIPE)
    def _(i):
        zero_v[i, :] = jnp.zeros((_DH,), jnp.float32)

    @pl.loop(0, _CB * _CHUNK)
    def _(i):
        ones_v[i, :] = jnp.full((_DH,), 1.0, jnp.float32)

    pltpu.async_copy(dst_hbm.at[w], dst_v, sem).wait()
    pltpu.sync_copy(zero_v, acc_sh.at[pl.ds(s * _STRIPE, _STRIPE)])
    plsc.subcore_barrier()

    @pl.loop(0, _KS)
    def _(j):
        pltpu.sync_copy(ones_v, acc_sh.at[dst_v.at[j]], add=True)

    plsc.subcore_barrier()
    pltpu.sync_copy(acc_sh.at[pl.ds(s * _STRIPE, _STRIPE)],
                    out_hbm.at[c, pl.ds(s * _STRIPE, _STRIPE)])


def _xw1_body(x_ref, w_ref, o_ref):
    o_ref[...] = jnp.dot(x_ref[...], w_ref[...],
                         preferred_element_type=jnp.float32)


def _tc_xw1(xp, w1):
    return pl.pallas_call(
        _xw1_body,
        grid=(_NPAD // _BM,),
        in_specs=[pl.BlockSpec((_BM, _DIN), lambda i: (i, 0)),
                  pl.BlockSpec((_DIN, _DH), lambda i: (0, 0))],
        out_specs=pl.BlockSpec((_BM, _DH), lambda i: (i, 0)),
        out_shape=jax.ShapeDtypeStruct((_NPAD, _DH), jnp.float32),
    )(xp, w1)


def _y1_body(degp_ref, xw_ref, dinv_ref, y1_ref):
    deg = degp_ref[0, :, 0:1] + degp_ref[1, :, 0:1] + 1.0
    dinv = lax.rsqrt(deg)
    dinv_ref[...] = dinv
    y1_ref[...] = xw_ref[...] * dinv


def _tc_y1(degp, xw1):
    return pl.pallas_call(
        _y1_body,
        grid=(_NPAD // _BM,),
        in_specs=[pl.BlockSpec((_NCORES, _BM, _DH), lambda i: (0, i, 0)),
                  pl.BlockSpec((_BM, _DH), lambda i: (i, 0))],
        out_specs=[pl.BlockSpec((_BM, 1), lambda i: (i, 0)),
                   pl.BlockSpec((_BM, _DH), lambda i: (i, 0))],
        out_shape=[jax.ShapeDtypeStruct((_NPAD, 1), jnp.float32),
                   jax.ShapeDtypeStruct((_NPAD, _DH), jnp.float32)],
    )(degp, xw1)


def _h_body(acc_ref, y1_ref, dinv_ref, b1_ref, y2_ref):
    a = acc_ref[0] + acc_ref[1] + y1_ref[...]
    h = jnp.maximum(a * dinv_ref[...] + b1_ref[...], 0.0)
    y2_ref[...] = h * dinv_ref[...]


def _tc_h(acc1, y1, dinv, b1r):
    return pl.pallas_call(
        _h_body,
        grid=(_NPAD // _BM,),
        in_specs=[pl.BlockSpec((_NCORES, _BM, _DH), lambda i: (0, i, 0)),
                  pl.BlockSpec((_BM, _DH), lambda i: (i, 0)),
                  pl.BlockSpec((_BM, 1), lambda i: (i, 0)),
                  pl.BlockSpec((1, _DH), lambda i: (0, 0))],
        out_specs=pl.BlockSpec((_BM, _DH), lambda i: (i, 0)),
        out_shape=jax.ShapeDtypeStruct((_NPAD, _DH), jnp.float32),
    )(acc1, y1, dinv, b1r)


def _out_body(acc_ref, y2_ref, dinv_ref, w2_ref, b2_ref, o_ref):
    p = (acc_ref[0] + acc_ref[1] + y2_ref[...]) * dinv_ref[...]
    o_ref[...] = jnp.dot(p, w2_ref[...],
                         preferred_element_type=jnp.float32) + b2_ref[...]


def _tc_out(acc2, y2, dinv, w2, b2r):
    return pl.pallas_call(
        _out_body,
        grid=(_NPAD // _BM,),
        in_specs=[pl.BlockSpec((_NCORES, _BM, _DH), lambda i: (0, i, 0)),
                  pl.BlockSpec((_BM, _DH), lambda i: (i, 0)),
                  pl.BlockSpec((_BM, 1), lambda i: (i, 0)),
                  pl.BlockSpec((_DH, _DOUT), lambda i: (0, 0)),
                  pl.BlockSpec((1, _DOUT), lambda i: (0, 0))],
        out_specs=pl.BlockSpec((_BM, _DOUT), lambda i: (i, 0)),
        out_shape=jax.ShapeDtypeStruct((_NPAD, _DOUT), jnp.float32),
    )(acc2, y2, dinv, w2, b2r)


def kernel(V, E, X, W1, b1, W2, b2):
    del V
    src = E[0]
    dst = E[1]
    fill = jnp.full((_NW * _KS * _CB * _CHUNK - _NE,), _N, jnp.int32)
    dstp = jnp.concatenate([dst, fill]).reshape(_NW, _KS, _CB * _CHUNK)
    srcp = jnp.concatenate(
        [jnp.concatenate([src, fill]).reshape(_NW, _KS, _CB * _CHUNK),
         jnp.full((_NW, _KSB - _KS, _CB * _CHUNK), _N, jnp.int32)], axis=1)
    xp = jnp.pad(X, ((0, _NPAD - _N), (0, 0)))

    degp = _deg(dstp)
    xw1 = _tc_xw1(xp, W1)
    dinv, y1 = _tc_y1(degp, xw1)
    acc1 = _prop(y1, srcp, dstp)
    y2 = _tc_h(acc1, y1, dinv, b1.reshape(1, _DH))
    acc2 = _prop(y2, srcp, dstp)
    out = _tc_out(acc2, y2, dinv, W2, b2.reshape(1, _DOUT))
    return out[:_N]


# speedup vs baseline: 1.5506x; 1.5506x over previous
import functools

import jax
import jax.numpy as jnp
from jax import lax
from jax.experimental import pallas as pl
from jax.experimental.pallas import tpu as pltpu
from jax.experimental.pallas import tpu_sc as plsc

_N = 10000
_NE = 320000
_DIN = 128
_DH = 16
_DOUT = 128

_NPAD = 10240
_STRIPE = _NPAD // 16
_NCORES = 2
_NSUB = 16
_NW = _NCORES * _NSUB
_CHUNK = 128
_NB = 8
_K = 80
_KBUF = _K + _NB

_BM = 2048

_vmesh = plsc.VectorSubcoreMesh(core_axis_name="c", subcore_axis_name="s")


@functools.partial(
    pl.kernel,
    mesh=_vmesh,
    out_type=jax.ShapeDtypeStruct((_NCORES, _NPAD, _DH), jnp.float32),
    scratch_types=[
        pltpu.VMEM((_KBUF, _CHUNK), jnp.int32),
        pltpu.VMEM((_K, _CHUNK), jnp.int32),
        pltpu.VMEM((_NB, _CHUNK, _DH), jnp.float32),
        pltpu.VMEM((_STRIPE, _DH), jnp.float32),
        pltpu.VMEM_SHARED((_NPAD, _DH), jnp.float32),
        pltpu.SemaphoreType.DMA((_NB,)),
    ],
    compiler_params=pltpu.CompilerParams(use_tc_tiling_on_sc=False),
)
def _prop(y_hbm, src_hbm, dst_hbm, out_hbm,
          src_v, dst_v, rows_v, zero_v, acc_sh, gsem):
    c = lax.axis_index("c")
    s = lax.axis_index("s")
    w = c * _NSUB + s

    @pl.loop(0, _STRIPE)
    def _(i):
        zero_v[i, :] = jnp.zeros((_DH,), jnp.float32)

    pltpu.async_copy(src_hbm.at[w], src_v, gsem.at[0]).wait()
    pltpu.async_copy(dst_hbm.at[w], dst_v, gsem.at[0]).wait()
    pltpu.sync_copy(zero_v, acc_sh.at[pl.ds(s * _STRIPE, _STRIPE)])
    plsc.subcore_barrier()

    for b in range(_NB):
        pltpu.async_copy(y_hbm.at[src_v.at[b]], rows_v.at[b], gsem.at[b])

    @pl.loop(0, _K, step=_NB)
    def _(j):
        for b in range(_NB):
            pltpu.make_async_copy(
                y_hbm.at[src_v.at[j + b]], rows_v.at[b], gsem.at[b]).wait()
            pltpu.sync_copy(rows_v.at[b], acc_sh.at[dst_v.at[j + b]], add=True)
            pltpu.async_copy(
                y_hbm.at[src_v.at[j + b + _NB]], rows_v.at[b], gsem.at[b])

    for b in range(_NB):
        pltpu.make_async_copy(
            y_hbm.at[src_v.at[_K + b]], rows_v.at[b], gsem.at[b]).wait()

    plsc.subcore_barrier()
    pltpu.sync_copy(acc_sh.at[pl.ds(s * _STRIPE, _STRIPE)],
                    out_hbm.at[c, pl.ds(s * _STRIPE, _STRIPE)])


@functools.partial(
    pl.kernel,
    mesh=_vmesh,
    out_type=jax.ShapeDtypeStruct((_NCORES, _NPAD, _DH), jnp.float32),
    scratch_types=[
        pltpu.VMEM((_K, _CHUNK), jnp.int32),
        pltpu.VMEM((_CHUNK, _DH), jnp.float32),
        pltpu.VMEM((_STRIPE, _DH), jnp.float32),
        pltpu.VMEM_SHARED((_NPAD, _DH), jnp.float32),
        pltpu.SemaphoreType.DMA,
    ],
    compiler_params=pltpu.CompilerParams(use_tc_tiling_on_sc=False),
)
def _deg(dst_hbm, out_hbm, dst_v, ones_v, zero_v, acc_sh, sem):
    c = lax.axis_index("c")
    s = lax.axis_index("s")
    w = c * _NSUB + s

    @pl.loop(0, _STRIPE)
    def _(i):
        zero_v[i, :] = jnp.zeros((_DH,), jnp.float32)

    @pl.loop(0, _CHUNK)
    def _(i):
        ones_v[i, :] = jnp.full((_DH,), 1.0, jnp.float32)

    pltpu.async_copy(dst_hbm.at[w], dst_v, sem).wait()
    pltpu.sync_copy(zero_v, acc_sh.at[pl.ds(s * _STRIPE, _STRIPE)])
    plsc.subcore_barrier()

    @pl.loop(0, _K)
    def _(j):
        pltpu.async_copy(ones_v, acc_sh.at[dst_v.at[j]], sem, add=True)

    @pl.loop(0, _K)
    def _(j):
        pltpu.make_async_copy(ones_v, acc_sh.at[dst_v.at[0]], sem).wait()

    plsc.subcore_barrier()
    pltpu.sync_copy(acc_sh.at[pl.ds(s * _STRIPE, _STRIPE)],
                    out_hbm.at[c, pl.ds(s * _STRIPE, _STRIPE)])


def _xw1_body(x_ref, w_ref, o_ref):
    o_ref[...] = jnp.dot(x_ref[...], w_ref[...],
                         preferred_element_type=jnp.float32)


def _tc_xw1(xp, w1):
    return pl.pallas_call(
        _xw1_body,
        grid=(_NPAD // _BM,),
        in_specs=[pl.BlockSpec((_BM, _DIN), lambda i: (i, 0)),
                  pl.BlockSpec((_DIN, _DH), lambda i: (0, 0))],
        out_specs=pl.BlockSpec((_BM, _DH), lambda i: (i, 0)),
        out_shape=jax.ShapeDtypeStruct((_NPAD, _DH), jnp.float32),
    )(xp, w1)


def _y1_body(degp_ref, xw_ref, dinv_ref, y1_ref):
    deg = degp_ref[0, :, 0:1] + degp_ref[1, :, 0:1] + 1.0
    dinv = lax.rsqrt(deg)
    dinv_ref[...] = dinv
    y1_ref[...] = xw_ref[...] * dinv


def _tc_y1(degp, xw1):
    return pl.pallas_call(
        _y1_body,
        grid=(_NPAD // _BM,),
        in_specs=[pl.BlockSpec((_NCORES, _BM, _DH), lambda i: (0, i, 0)),
                  pl.BlockSpec((_BM, _DH), lambda i: (i, 0))],
        out_specs=[pl.BlockSpec((_BM, 1), lambda i: (i, 0)),
                   pl.BlockSpec((_BM, _DH), lambda i: (i, 0))],
        out_shape=[jax.ShapeDtypeStruct((_NPAD, 1), jnp.float32),
                   jax.ShapeDtypeStruct((_NPAD, _DH), jnp.float32)],
    )(degp, xw1)


def _h_body(acc_ref, y1_ref, dinv_ref, b1_ref, y2_ref):
    a = acc_ref[0] + acc_ref[1] + y1_ref[...]
    h = jnp.maximum(a * dinv_ref[...] + b1_ref[...], 0.0)
    y2_ref[...] = h * dinv_ref[...]


def _tc_h(acc1, y1, dinv, b1r):
    return pl.pallas_call(
        _h_body,
        grid=(_NPAD // _BM,),
        in_specs=[pl.BlockSpec((_NCORES, _BM, _DH), lambda i: (0, i, 0)),
                  pl.BlockSpec((_BM, _DH), lambda i: (i, 0)),
                  pl.BlockSpec((_BM, 1), lambda i: (i, 0)),
                  pl.BlockSpec((1, _DH), lambda i: (0, 0))],
        out_specs=pl.BlockSpec((_BM, _DH), lambda i: (i, 0)),
        out_shape=jax.ShapeDtypeStruct((_NPAD, _DH), jnp.float32),
    )(acc1, y1, dinv, b1r)


def _out_body(acc_ref, y2_ref, dinv_ref, w2_ref, b2_ref, o_ref):
    p = (acc_ref[0] + acc_ref[1] + y2_ref[...]) * dinv_ref[...]
    o_ref[...] = jnp.dot(p, w2_ref[...],
                         preferred_element_type=jnp.float32) + b2_ref[...]


def _tc_out(acc2, y2, dinv, w2, b2r):
    return pl.pallas_call(
        _out_body,
        grid=(_NPAD // _BM,),
        in_specs=[pl.BlockSpec((_NCORES, _BM, _DH), lambda i: (0, i, 0)),
                  pl.BlockSpec((_BM, _DH), lambda i: (i, 0)),
                  pl.BlockSpec((_BM, 1), lambda i: (i, 0)),
                  pl.BlockSpec((_DH, _DOUT), lambda i: (0, 0)),
                  pl.BlockSpec((1, _DOUT), lambda i: (0, 0))],
        out_specs=pl.BlockSpec((_BM, _DOUT), lambda i: (i, 0)),
        out_shape=jax.ShapeDtypeStruct((_NPAD, _DOUT), jnp.float32),
    )(acc2, y2, dinv, w2, b2r)


def kernel(V, E, X, W1, b1, W2, b2):
    del V
    src = E[0]
    dst = E[1]
    fill = jnp.full((_NW * _K * _CHUNK - _NE,), _N, jnp.int32)
    dstp = jnp.concatenate([dst, fill]).reshape(_NW, _K, _CHUNK)
    srcp = jnp.concatenate(
        [jnp.concatenate([src, fill]).reshape(_NW, _K, _CHUNK),
         jnp.full((_NW, _KBUF - _K, _CHUNK), _N, jnp.int32)], axis=1)
    xp = jnp.pad(X, ((0, _NPAD - _N), (0, 0)))

    degp = _deg(dstp)
    xw1 = _tc_xw1(xp, W1)
    dinv, y1 = _tc_y1(degp, xw1)
    acc1 = _prop(y1, srcp, dstp)
    y2 = _tc_h(acc1, y1, dinv, b1.reshape(1, _DH))
    acc2 = _prop(y2, srcp, dstp)
    out = _tc_out(acc2, y2, dinv, W2, b2.reshape(1, _DOUT))
    return out[:_N]


# speedup vs baseline: 2.1192x; 1.3667x over previous
import functools

import jax
import jax.numpy as jnp
from jax import lax
from jax.experimental import pallas as pl
from jax.experimental.pallas import tpu as pltpu
from jax.experimental.pallas import tpu_sc as plsc

_N = 10000
_NE = 320000
_DIN = 128
_DH = 16
_DOUT = 128

_NPAD = 10240
_STRIPE = _NPAD // 16
_NCORES = 2
_NSUB = 16
_NW = _NCORES * _NSUB
_CHUNK = 128
_NB = 4
_K = 80
_KBUF = _K + _NB

_BM = 2048

_vmesh = plsc.VectorSubcoreMesh(core_axis_name="c", subcore_axis_name="s")


@functools.partial(
    pl.kernel,
    mesh=_vmesh,
    out_type=jax.ShapeDtypeStruct((_NCORES, _NPAD, _DH), jnp.float32),
    scratch_types=[
        pltpu.VMEM((_KBUF, _CHUNK), jnp.int32),
        pltpu.VMEM((_K, _CHUNK), jnp.int32),
        pltpu.VMEM((_NB, _CHUNK, _DH), jnp.float32),
        pltpu.VMEM((_STRIPE, _DH), jnp.float32),
        pltpu.VMEM_SHARED((_NPAD, _DH), jnp.float32),
        pltpu.SemaphoreType.DMA((_NB,)),
    ],
    compiler_params=pltpu.CompilerParams(use_tc_tiling_on_sc=False),
)
def _prop(y_hbm, src_hbm, dst_hbm, out_hbm,
          src_v, dst_v, rows_v, zero_v, acc_sh, gsem):
    c = lax.axis_index("c")
    s = lax.axis_index("s")
    w = c * _NSUB + s

    @pl.loop(0, _STRIPE)
    def _(i):
        zero_v[i, :] = jnp.zeros((_DH,), jnp.float32)

    pltpu.async_copy(src_hbm.at[w], src_v, gsem.at[0]).wait()
    pltpu.async_copy(dst_hbm.at[w], dst_v, gsem.at[0]).wait()
    pltpu.sync_copy(zero_v, acc_sh.at[pl.ds(s * _STRIPE, _STRIPE)])
    plsc.subcore_barrier()

    for b in range(_NB):
        pltpu.async_copy(y_hbm.at[src_v.at[b]], rows_v.at[b], gsem.at[b])

    @pl.loop(0, _K, step=_NB)
    def _(j):
        for b in range(_NB):
            pltpu.make_async_copy(
                y_hbm.at[src_v.at[j + b]], rows_v.at[b], gsem.at[b]).wait()
            pltpu.sync_copy(rows_v.at[b], acc_sh.at[dst_v.at[j + b]], add=True)
            pltpu.async_copy(
                y_hbm.at[src_v.at[j + b + _NB]], rows_v.at[b], gsem.at[b])

    for b in range(_NB):
        pltpu.make_async_copy(
            y_hbm.at[src_v.at[_K + b]], rows_v.at[b], gsem.at[b]).wait()

    plsc.subcore_barrier()
    pltpu.sync_copy(acc_sh.at[pl.ds(s * _STRIPE, _STRIPE)],
                    out_hbm.at[c, pl.ds(s * _STRIPE, _STRIPE)])


@functools.partial(
    pl.kernel,
    mesh=_vmesh,
    out_type=jax.ShapeDtypeStruct((_NCORES, _NPAD, _DH), jnp.float32),
    scratch_types=[
        pltpu.VMEM((_K, _CHUNK), jnp.int32),
        pltpu.VMEM((_CHUNK, _DH), jnp.float32),
        pltpu.VMEM((_STRIPE, _DH), jnp.float32),
        pltpu.VMEM_SHARED((_NPAD, _DH), jnp.float32),
        pltpu.SemaphoreType.DMA,
    ],
    compiler_params=pltpu.CompilerParams(use_tc_tiling_on_sc=False),
)
def _deg(dst_hbm, out_hbm, dst_v, ones_v, zero_v, acc_sh, sem):
    c = lax.axis_index("c")
    s = lax.axis_index("s")
    w = c * _NSUB + s

    @pl.loop(0, _STRIPE)
    def _(i):
        zero_v[i, :] = jnp.zeros((_DH,), jnp.float32)

    @pl.loop(0, _CHUNK)
    def _(i):
        ones_v[i, :] = jnp.full((_DH,), 1.0, jnp.float32)

    pltpu.async_copy(dst_hbm.at[w], dst_v, sem).wait()
    pltpu.sync_copy(zero_v, acc_sh.at[pl.ds(s * _STRIPE, _STRIPE)])
    plsc.subcore_barrier()

    @pl.loop(0, _K)
    def _(j):
        pltpu.async_copy(ones_v, acc_sh.at[dst_v.at[j]], sem, add=True)

    @pl.loop(0, _K)
    def _(j):
        pltpu.make_async_copy(ones_v, acc_sh.at[dst_v.at[0]], sem).wait()

    plsc.subcore_barrier()
    pltpu.sync_copy(acc_sh.at[pl.ds(s * _STRIPE, _STRIPE)],
                    out_hbm.at[c, pl.ds(s * _STRIPE, _STRIPE)])


def _xw1_body(x_ref, w_ref, o_ref):
    o_ref[...] = jnp.dot(x_ref[...], w_ref[...],
                         preferred_element_type=jnp.float32)


def _tc_xw1(xp, w1):
    return pl.pallas_call(
        _xw1_body,
        grid=(_NPAD // _BM,),
        in_specs=[pl.BlockSpec((_BM, _DIN), lambda i: (i, 0)),
                  pl.BlockSpec((_DIN, _DH), lambda i: (0, 0))],
        out_specs=pl.BlockSpec((_BM, _DH), lambda i: (i, 0)),
        out_shape=jax.ShapeDtypeStruct((_NPAD, _DH), jnp.float32),
    )(xp, w1)


def _y1_body(degp_ref, xw_ref, dinv_ref, y1_ref):
    deg = degp_ref[0, :, 0:1] + degp_ref[1, :, 0:1] + 1.0
    dinv = lax.rsqrt(deg)
    dinv_ref[...] = dinv
    y1_ref[...] = xw_ref[...] * dinv


def _tc_y1(degp, xw1):
    return pl.pallas_call(
        _y1_body,
        grid=(_NPAD // _BM,),
        in_specs=[pl.BlockSpec((_NCORES, _BM, _DH), lambda i: (0, i, 0)),
                  pl.BlockSpec((_BM, _DH), lambda i: (i, 0))],
        out_specs=[pl.BlockSpec((_BM, 1), lambda i: (i, 0)),
                   pl.BlockSpec((_BM, _DH), lambda i: (i, 0))],
        out_shape=[jax.ShapeDtypeStruct((_NPAD, 1), jnp.float32),
                   jax.ShapeDtypeStruct((_NPAD, _DH), jnp.float32)],
    )(degp, xw1)


def _h_body(acc_ref, y1_ref, dinv_ref, b1_ref, y2_ref):
    a = acc_ref[0] + acc_ref[1] + y1_ref[...]
    h = jnp.maximum(a * dinv_ref[...] + b1_ref[...], 0.0)
    y2_ref[...] = h * dinv_ref[...]


def _tc_h(acc1, y1, dinv, b1r):
    return pl.pallas_call(
        _h_body,
        grid=(_NPAD // _BM,),
        in_specs=[pl.BlockSpec((_NCORES, _BM, _DH), lambda i: (0, i, 0)),
                  pl.BlockSpec((_BM, _DH), lambda i: (i, 0)),
                  pl.BlockSpec((_BM, 1), lambda i: (i, 0)),
                  pl.BlockSpec((1, _DH), lambda i: (0, 0))],
        out_specs=pl.BlockSpec((_BM, _DH), lambda i: (i, 0)),
        out_shape=jax.ShapeDtypeStruct((_NPAD, _DH), jnp.float32),
    )(acc1, y1, dinv, b1r)


def _out_body(acc_ref, y2_ref, dinv_ref, w2_ref, b2_ref, o_ref):
    p = (acc_ref[0] + acc_ref[1] + y2_ref[...]) * dinv_ref[...]
    o_ref[...] = jnp.dot(p, w2_ref[...],
                         preferred_element_type=jnp.float32) + b2_ref[...]


def _tc_out(acc2, y2, dinv, w2, b2r):
    return pl.pallas_call(
        _out_body,
        grid=(_NPAD // _BM,),
        in_specs=[pl.BlockSpec((_NCORES, _BM, _DH), lambda i: (0, i, 0)),
                  pl.BlockSpec((_BM, _DH), lambda i: (i, 0)),
                  pl.BlockSpec((_BM, 1), lambda i: (i, 0)),
                  pl.BlockSpec((_DH, _DOUT), lambda i: (0, 0)),
                  pl.BlockSpec((1, _DOUT), lambda i: (0, 0))],
        out_specs=pl.BlockSpec((_BM, _DOUT), lambda i: (i, 0)),
        out_shape=jax.ShapeDtypeStruct((_NPAD, _DOUT), jnp.float32),
    )(acc2, y2, dinv, w2, b2r)


def kernel(V, E, X, W1, b1, W2, b2):
    del V
    src = E[0]
    dst = E[1]
    fill = jnp.full((_NW * _K * _CHUNK - _NE,), _N, jnp.int32)
    dstp = jnp.concatenate([dst, fill]).reshape(_NW, _K, _CHUNK)
    srcp = jnp.concatenate(
        [jnp.concatenate([src, fill]).reshape(_NW, _K, _CHUNK),
         jnp.full((_NW, _KBUF - _K, _CHUNK), _N, jnp.int32)], axis=1)
    xp = jnp.pad(X, ((0, _NPAD - _N), (0, 0)))

    degp = _deg(dstp)
    xw1 = _tc_xw1(xp, W1)
    dinv, y1 = _tc_y1(degp, xw1)
    acc1 = _prop(y1, srcp, dstp)
    y2 = _tc_h(acc1, y1, dinv, b1.reshape(1, _DH))
    acc2 = _prop(y2, srcp, dstp)
    out = _tc_out(acc2, y2, dinv, W2, b2.reshape(1, _DOUT))
    return out[:_N]


# speedup vs baseline: 4.9520x; 2.3367x over previous
import functools

import jax
import jax.numpy as jnp
from jax import lax
from jax.experimental import pallas as pl
from jax.experimental.pallas import tpu as pltpu
from jax.experimental.pallas import tpu_sc as plsc

_N = 10000
_NE = 320000
_DIN = 128
_DH = 16
_DOUT = 128

_NPAD = 10240
_STRIPE = _NPAD // 16
_NCORES = 2
_NSUB = 16
_NW = _NCORES * _NSUB
_CHUNK = 128
_NB = 4
_K = 80
_KBUF = _K + _NB

_BM = 2048

_vmesh = plsc.VectorSubcoreMesh(core_axis_name="c", subcore_axis_name="s")


@functools.partial(
    pl.kernel,
    mesh=_vmesh,
    out_type=jax.ShapeDtypeStruct((_NCORES, _NPAD, _DH), jnp.float32),
    scratch_types=[
        pltpu.VMEM((_K, _CHUNK), jnp.int32),
        pltpu.VMEM((_K, _CHUNK), jnp.int32),
        pltpu.VMEM((2, _CHUNK, _DH), jnp.float32),
        pltpu.VMEM((_STRIPE, _DH), jnp.float32),
        pltpu.VMEM_SHARED((_NPAD, _DH), jnp.float32),
        pltpu.VMEM_SHARED((_NPAD, _DH), jnp.float32),
        pltpu.SemaphoreType.DMA((2,)),
        pltpu.SemaphoreType.DMA,
    ],
    compiler_params=pltpu.CompilerParams(use_tc_tiling_on_sc=False),
)
def _prop(y_hbm, src_hbm, dst_hbm, out_hbm,
          src_v, dst_v, rows_v, zero_v, y_sh, acc_sh, ssem, sem):
    c = lax.axis_index("c")
    s = lax.axis_index("s")
    w = c * _NSUB + s

    @pl.loop(0, _STRIPE)
    def _(i):
        zero_v[i, :] = jnp.zeros((_DH,), jnp.float32)

    pltpu.async_copy(y_hbm.at[pl.ds(s * _STRIPE, _STRIPE)],
                     y_sh.at[pl.ds(s * _STRIPE, _STRIPE)], sem)
    pltpu.async_copy(src_hbm.at[w], src_v, sem)
    pltpu.async_copy(dst_hbm.at[w], dst_v, sem)
    pltpu.sync_copy(zero_v, acc_sh.at[pl.ds(s * _STRIPE, _STRIPE)])
    pltpu.make_async_copy(y_hbm.at[pl.ds(s * _STRIPE, _STRIPE)],
                          y_sh.at[pl.ds(s * _STRIPE, _STRIPE)], sem).wait()
    pltpu.make_async_copy(src_hbm.at[w], src_v, sem).wait()
    pltpu.make_async_copy(dst_hbm.at[w], dst_v, sem).wait()
    plsc.subcore_barrier()

    for b in range(2):
        pltpu.sync_copy(y_sh.at[src_v.at[b]], rows_v.at[b])
        pltpu.async_copy(rows_v.at[b], acc_sh.at[dst_v.at[b]], ssem.at[b],
                         add=True)

    @pl.loop(2, _K, step=2)
    def _(j):
        for b in range(2):
            pltpu.make_async_copy(rows_v.at[b], acc_sh.at[dst_v.at[0]],
                                  ssem.at[b]).wait()
            pltpu.sync_copy(y_sh.at[src_v.at[j + b]], rows_v.at[b])
            pltpu.async_copy(rows_v.at[b], acc_sh.at[dst_v.at[j + b]],
                             ssem.at[b], add=True)

    for b in range(2):
        pltpu.make_async_copy(rows_v.at[b], acc_sh.at[dst_v.at[0]],
                              ssem.at[b]).wait()

    plsc.subcore_barrier()
    pltpu.sync_copy(acc_sh.at[pl.ds(s * _STRIPE, _STRIPE)],
                    out_hbm.at[c, pl.ds(s * _STRIPE, _STRIPE)])


@functools.partial(
    pl.kernel,
    mesh=_vmesh,
    out_type=jax.ShapeDtypeStruct((_NCORES, _NPAD, _DH), jnp.float32),
    scratch_types=[
        pltpu.VMEM((_K, _CHUNK), jnp.int32),
        pltpu.VMEM((_CHUNK, _DH), jnp.float32),
        pltpu.VMEM((_STRIPE, _DH), jnp.float32),
        pltpu.VMEM_SHARED((_NPAD, _DH), jnp.float32),
        pltpu.SemaphoreType.DMA,
    ],
    compiler_params=pltpu.CompilerParams(use_tc_tiling_on_sc=False),
)
def _deg(dst_hbm, out_hbm, dst_v, ones_v, zero_v, acc_sh, sem):
    c = lax.axis_index("c")
    s = lax.axis_index("s")
    w = c * _NSUB + s

    @pl.loop(0, _STRIPE)
    def _(i):
        zero_v[i, :] = jnp.zeros((_DH,), jnp.float32)

    @pl.loop(0, _CHUNK)
    def _(i):
        ones_v[i, :] = jnp.full((_DH,), 1.0, jnp.float32)

    pltpu.async_copy(dst_hbm.at[w], dst_v, sem).wait()
    pltpu.sync_copy(zero_v, acc_sh.at[pl.ds(s * _STRIPE, _STRIPE)])
    plsc.subcore_barrier()

    @pl.loop(0, _K)
    def _(j):
        pltpu.async_copy(ones_v, acc_sh.at[dst_v.at[j]], sem, add=True)

    @pl.loop(0, _K)
    def _(j):
        pltpu.make_async_copy(ones_v, acc_sh.at[dst_v.at[0]], sem).wait()

    plsc.subcore_barrier()
    pltpu.sync_copy(acc_sh.at[pl.ds(s * _STRIPE, _STRIPE)],
                    out_hbm.at[c, pl.ds(s * _STRIPE, _STRIPE)])


def _xw1_body(x_ref, w_ref, o_ref):
    o_ref[...] = jnp.dot(x_ref[...], w_ref[...],
                         preferred_element_type=jnp.float32)


def _tc_xw1(xp, w1):
    return pl.pallas_call(
        _xw1_body,
        grid=(_NPAD // _BM,),
        in_specs=[pl.BlockSpec((_BM, _DIN), lambda i: (i, 0)),
                  pl.BlockSpec((_DIN, _DH), lambda i: (0, 0))],
        out_specs=pl.BlockSpec((_BM, _DH), lambda i: (i, 0)),
        out_shape=jax.ShapeDtypeStruct((_NPAD, _DH), jnp.float32),
    )(xp, w1)


def _y1_body(degp_ref, xw_ref, dinv_ref, y1_ref):
    deg = degp_ref[0, :, 0:1] + degp_ref[1, :, 0:1] + 1.0
    dinv = lax.rsqrt(deg)
    dinv_ref[...] = dinv
    y1_ref[...] = xw_ref[...] * dinv


def _tc_y1(degp, xw1):
    return pl.pallas_call(
        _y1_body,
        grid=(_NPAD // _BM,),
        in_specs=[pl.BlockSpec((_NCORES, _BM, _DH), lambda i: (0, i, 0)),
                  pl.BlockSpec((_BM, _DH), lambda i: (i, 0))],
        out_specs=[pl.BlockSpec((_BM, 1), lambda i: (i, 0)),
                   pl.BlockSpec((_BM, _DH), lambda i: (i, 0))],
        out_shape=[jax.ShapeDtypeStruct((_NPAD, 1), jnp.float32),
                   jax.ShapeDtypeStruct((_NPAD, _DH), jnp.float32)],
    )(degp, xw1)


def _h_body(acc_ref, y1_ref, dinv_ref, b1_ref, y2_ref):
    a = acc_ref[0] + acc_ref[1] + y1_ref[...]
    h = jnp.maximum(a * dinv_ref[...] + b1_ref[...], 0.0)
    y2_ref[...] = h * dinv_ref[...]


def _tc_h(acc1, y1, dinv, b1r):
    return pl.pallas_call(
        _h_body,
        grid=(_NPAD // _BM,),
        in_specs=[pl.BlockSpec((_NCORES, _BM, _DH), lambda i: (0, i, 0)),
                  pl.BlockSpec((_BM, _DH), lambda i: (i, 0)),
                  pl.BlockSpec((_BM, 1), lambda i: (i, 0)),
                  pl.BlockSpec((1, _DH), lambda i: (0, 0))],
        out_specs=pl.BlockSpec((_BM, _DH), lambda i: (i, 0)),
        out_shape=jax.ShapeDtypeStruct((_NPAD, _DH), jnp.float32),
    )(acc1, y1, dinv, b1r)


def _out_body(acc_ref, y2_ref, dinv_ref, w2_ref, b2_ref, o_ref):
    p = (acc_ref[0] + acc_ref[1] + y2_ref[...]) * dinv_ref[...]
    o_ref[...] = jnp.dot(p, w2_ref[...],
                         preferred_element_type=jnp.float32) + b2_ref[...]


def _tc_out(acc2, y2, dinv, w2, b2r):
    return pl.pallas_call(
        _out_body,
        grid=(_NPAD // _BM,),
        in_specs=[pl.BlockSpec((_NCORES, _BM, _DH), lambda i: (0, i, 0)),
                  pl.BlockSpec((_BM, _DH), lambda i: (i, 0)),
                  pl.BlockSpec((_BM, 1), lambda i: (i, 0)),
                  pl.BlockSpec((_DH, _DOUT), lambda i: (0, 0)),
                  pl.BlockSpec((1, _DOUT), lambda i: (0, 0))],
        out_specs=pl.BlockSpec((_BM, _DOUT), lambda i: (i, 0)),
        out_shape=jax.ShapeDtypeStruct((_NPAD, _DOUT), jnp.float32),
    )(acc2, y2, dinv, w2, b2r)


def kernel(V, E, X, W1, b1, W2, b2):
    del V
    src = E[0]
    dst = E[1]
    fill = jnp.full((_NW * _K * _CHUNK - _NE,), _N, jnp.int32)
    dstp = jnp.concatenate([dst, fill]).reshape(_NW, _K, _CHUNK)
    srcp = jnp.concatenate([src, fill]).reshape(_NW, _K, _CHUNK)
    xp = jnp.pad(X, ((0, _NPAD - _N), (0, 0)))

    degp = _deg(dstp)
    xw1 = _tc_xw1(xp, W1)
    dinv, y1 = _tc_y1(degp, xw1)
    acc1 = _prop(y1, srcp, dstp)
    y2 = _tc_h(acc1, y1, dinv, b1.reshape(1, _DH))
    acc2 = _prop(y2, srcp, dstp)
    out = _tc_out(acc2, y2, dinv, W2, b2.reshape(1, _DOUT))
    return out[:_N]


# speedup vs baseline: 6.3851x; 1.2894x over previous
import functools

import jax
import jax.numpy as jnp
from jax import lax
from jax.experimental import pallas as pl
from jax.experimental.pallas import tpu as pltpu
from jax.experimental.pallas import tpu_sc as plsc

_N = 10000
_NE = 320000
_DIN = 128
_DH = 16
_DOUT = 128

_NPAD = 10240
_STRIPE = _NPAD // 16
_NCORES = 2
_NSUB = 16
_NW = _NCORES * _NSUB
_CHUNK = 128
_NB = 4
_K = 80
_KBUF = _K + _NB

_BM = 2048
_BM8 = _BM // 8
_NP8 = _NPAD // 8
_STR8 = _STRIPE // 8

_vmesh = plsc.VectorSubcoreMesh(core_axis_name="c", subcore_axis_name="s")


@functools.partial(
    pl.kernel,
    mesh=_vmesh,
    out_type=jax.ShapeDtypeStruct((_NCORES, _NP8, 128), jnp.float32),
    scratch_types=[
        pltpu.VMEM((_K, _CHUNK), jnp.int32),
        pltpu.VMEM((_K, _CHUNK), jnp.int32),
        pltpu.VMEM((2, _CHUNK, _DH), jnp.float32),
        pltpu.VMEM((_STR8, 128), jnp.float32),
        pltpu.VMEM((_STRIPE, _DH), jnp.float32),
        pltpu.VMEM_SHARED((_NPAD, _DH), jnp.float32),
        pltpu.VMEM_SHARED((_NPAD, _DH), jnp.float32),
        pltpu.SemaphoreType.DMA((2,)),
        pltpu.SemaphoreType.DMA,
    ],
    compiler_params=pltpu.CompilerParams(use_tc_tiling_on_sc=False),
)
def _prop(y_hbm, src_hbm, dst_hbm, out_hbm,
          src_v, dst_v, rows_v, pbuf_v, qbuf_v, y_sh, acc_sh, ssem, sem):
    c = lax.axis_index("c")
    s = lax.axis_index("s")
    w = c * _NSUB + s

    pltpu.async_copy(y_hbm.at[pl.ds(s * _STR8, _STR8)], pbuf_v, sem)
    pltpu.async_copy(src_hbm.at[w], src_v, sem)
    pltpu.async_copy(dst_hbm.at[w], dst_v, sem)
    pltpu.make_async_copy(y_hbm.at[pl.ds(s * _STR8, _STR8)], pbuf_v, sem).wait()

    @pl.loop(0, _STR8)
    def _(r):
        for k in range(8):
            qbuf_v[r * 8 + k, :] = pbuf_v[r, pl.ds(k * _DH, _DH)]

    pltpu.async_copy(qbuf_v, y_sh.at[pl.ds(s * _STRIPE, _STRIPE)], sem)

    @pl.loop(0, _STRIPE)
    def _(i):
        qbuf_v[i, :] = jnp.zeros((_DH,), jnp.float32)

    pltpu.make_async_copy(qbuf_v, y_sh.at[pl.ds(s * _STRIPE, _STRIPE)],
                          sem).wait()
    pltpu.sync_copy(qbuf_v, acc_sh.at[pl.ds(s * _STRIPE, _STRIPE)])
    pltpu.make_async_copy(src_hbm.at[w], src_v, sem).wait()
    pltpu.make_async_copy(dst_hbm.at[w], dst_v, sem).wait()
    plsc.subcore_barrier()

    for b in range(2):
        pltpu.sync_copy(y_sh.at[src_v.at[b]], rows_v.at[b])
        pltpu.async_copy(rows_v.at[b], acc_sh.at[dst_v.at[b]], ssem.at[b],
                         add=True)

    @pl.loop(2, _K, step=2)
    def _(j):
        for b in range(2):
            pltpu.make_async_copy(rows_v.at[b], acc_sh.at[dst_v.at[0]],
                                  ssem.at[b]).wait()
            pltpu.sync_copy(y_sh.at[src_v.at[j + b]], rows_v.at[b])
            pltpu.async_copy(rows_v.at[b], acc_sh.at[dst_v.at[j + b]],
                             ssem.at[b], add=True)

    for b in range(2):
        pltpu.make_async_copy(rows_v.at[b], acc_sh.at[dst_v.at[0]],
                              ssem.at[b]).wait()

    plsc.subcore_barrier()
    pltpu.sync_copy(acc_sh.at[pl.ds(s * _STRIPE, _STRIPE)], qbuf_v)

    @pl.loop(0, _STR8)
    def _(r):
        for k in range(8):
            pbuf_v[r, pl.ds(k * _DH, _DH)] = qbuf_v[r * 8 + k, :]

    pltpu.sync_copy(pbuf_v, out_hbm.at[c, pl.ds(s * _STR8, _STR8)])


@functools.partial(
    pl.kernel,
    mesh=_vmesh,
    out_type=jax.ShapeDtypeStruct((_NCORES, _NP8, 128), jnp.float32),
    scratch_types=[
        pltpu.VMEM((_K, _CHUNK), jnp.int32),
        pltpu.VMEM((_CHUNK, _DH), jnp.float32),
        pltpu.VMEM((_STR8, 128), jnp.float32),
        pltpu.VMEM((_STRIPE, _DH), jnp.float32),
        pltpu.VMEM_SHARED((_NPAD, _DH), jnp.float32),
        pltpu.SemaphoreType.DMA,
    ],
    compiler_params=pltpu.CompilerParams(use_tc_tiling_on_sc=False),
)
def _deg(dst_hbm, out_hbm, dst_v, ones_v, pbuf_v, qbuf_v, acc_sh, sem):
    c = lax.axis_index("c")
    s = lax.axis_index("s")
    w = c * _NSUB + s

    @pl.loop(0, _STRIPE)
    def _(i):
        qbuf_v[i, :] = jnp.zeros((_DH,), jnp.float32)

    @pl.loop(0, _CHUNK)
    def _(i):
        ones_v[i, :] = jnp.full((_DH,), 1.0, jnp.float32)

    pltpu.async_copy(dst_hbm.at[w], dst_v, sem).wait()
    pltpu.sync_copy(qbuf_v, acc_sh.at[pl.ds(s * _STRIPE, _STRIPE)])
    plsc.subcore_barrier()

    @pl.loop(0, _K)
    def _(j):
        pltpu.async_copy(ones_v, acc_sh.at[dst_v.at[j]], sem, add=True)

    @pl.loop(0, _K)
    def _(j):
        pltpu.make_async_copy(ones_v, acc_sh.at[dst_v.at[0]], sem).wait()

    plsc.subcore_barrier()
    pltpu.sync_copy(acc_sh.at[pl.ds(s * _STRIPE, _STRIPE)], qbuf_v)

    @pl.loop(0, _STR8)
    def _(r):
        for k in range(8):
            pbuf_v[r, pl.ds(k * _DH, _DH)] = qbuf_v[r * 8 + k, :]

    pltpu.sync_copy(pbuf_v, out_hbm.at[c, pl.ds(s * _STR8, _STR8)])


def _y1_body(degp_ref, x_ref, w_ref, dinvp_ref, y1p_ref):
    deg = degp_ref[0] + degp_ref[1] + 1.0
    dinvp = lax.rsqrt(deg)
    dinvp_ref[...] = dinvp
    xw = jnp.dot(x_ref[...], w_ref[...], preferred_element_type=jnp.float32)
    xwp = jnp.concatenate(
        [xw[_NP8 * k:_NP8 * (k + 1), :] for k in range(8)], axis=1)
    y1p_ref[...] = xwp * dinvp


def _tc_y1(degp, xpermp, w1):
    return pl.pallas_call(
        _y1_body,
        grid=(1,),
        in_specs=[pl.BlockSpec((_NCORES, _NP8, 128), lambda i: (0, 0, 0)),
                  pl.BlockSpec((_NPAD, _DIN), lambda i: (0, 0)),
                  pl.BlockSpec((_DIN, _DH), lambda i: (0, 0))],
        out_specs=[pl.BlockSpec((_NP8, 128), lambda i: (0, 0)),
                   pl.BlockSpec((_NP8, 128), lambda i: (0, 0))],
        out_shape=[jax.ShapeDtypeStruct((_NP8, 128), jnp.float32),
                   jax.ShapeDtypeStruct((_NP8, 128), jnp.float32)],
    )(degp, xpermp, w1)


def _h_body(acc_ref, y1p_ref, dinvp_ref, b1p_ref, y2p_ref):
    a = acc_ref[0] + acc_ref[1] + y1p_ref[...]
    h = jnp.maximum(a * dinvp_ref[...] + b1p_ref[...], 0.0)
    y2p_ref[...] = h * dinvp_ref[...]


def _tc_h(acc1, y1p, dinvp, b1p):
    return pl.pallas_call(
        _h_body,
        grid=(_NPAD // _BM,),
        in_specs=[pl.BlockSpec((_NCORES, _BM8, 128), lambda i: (0, i, 0)),
                  pl.BlockSpec((_BM8, 128), lambda i: (i, 0)),
                  pl.BlockSpec((_BM8, 128), lambda i: (i, 0)),
                  pl.BlockSpec((1, 128), lambda i: (0, 0))],
        out_specs=pl.BlockSpec((_BM8, 128), lambda i: (i, 0)),
        out_shape=jax.ShapeDtypeStruct((_NP8, 128), jnp.float32),
    )(acc1, y1p, dinvp, b1p)


def _out_body(acc_ref, y2p_ref, dinvp_ref, w2_ref, b2_ref, o_ref):
    pp = (acc_ref[0] + acc_ref[1] + y2p_ref[...]) * dinvp_ref[...]
    p = jnp.concatenate(
        [pp[:, _DH * k:_DH * (k + 1)] for k in range(8)], axis=0)
    o_ref[...] = jnp.dot(p, w2_ref[...],
                         preferred_element_type=jnp.float32) + b2_ref[...]


def _tc_out(acc2, y2p, dinvp, w2, b2r):
    return pl.pallas_call(
        _out_body,
        grid=(1,),
        in_specs=[pl.BlockSpec((_NCORES, _NP8, 128), lambda i: (0, 0, 0)),
                  pl.BlockSpec((_NP8, 128), lambda i: (0, 0)),
                  pl.BlockSpec((_NP8, 128), lambda i: (0, 0)),
                  pl.BlockSpec((_DH, _DOUT), lambda i: (0, 0)),
                  pl.BlockSpec((1, _DOUT), lambda i: (0, 0))],
        out_specs=pl.BlockSpec((_NPAD, _DOUT), lambda i: (0, 0)),
        out_shape=jax.ShapeDtypeStruct((_NPAD, _DOUT), jnp.float32),
    )(acc2, y2p, dinvp, w2, b2r)


def kernel(V, E, X, W1, b1, W2, b2):
    del V
    src = E[0]
    dst = E[1]
    fill = jnp.full((_NW * _K * _CHUNK - _NE,), _N, jnp.int32)
    dstp = jnp.concatenate([dst, fill]).reshape(_NW, _K, _CHUNK)
    srcp = jnp.concatenate([src, fill]).reshape(_NW, _K, _CHUNK)
    xp = jnp.pad(X, ((0, _NPAD - _N), (0, 0)))
    xperm = xp.reshape(_NP8, 8, _DIN).transpose(1, 0, 2).reshape(_NPAD, _DIN)
    b1p = jnp.tile(b1, 8).reshape(1, 128)

    degp = _deg(dstp)
    dinvp, y1p = _tc_y1(degp, xperm, W1)
    acc1p = _prop(y1p, srcp, dstp)
    y2p = _tc_h(acc1p, y1p, dinvp, b1p)
    acc2p = _prop(y2p, srcp, dstp)
    outperm = _tc_out(acc2p, y2p, dinvp, W2, b2.reshape(1, _DOUT))
    out = outperm.reshape(8, _NP8, _DOUT).transpose(1, 0, 2).reshape(
        _NPAD, _DOUT)
    return out[:_N]
